# baseline (speedup 1.0000x reference)
import functools

import jax
import jax.numpy as jnp
from jax.experimental import pallas as pl
from jax.experimental.pallas import tpu as pltpu

_INTERPRET = False


def _silu(v):
    return v * jax.nn.sigmoid(v)


def _layer_body(n, bi, bj, h_ref, x_ref, wa_ref, wb_ref, wd_ref, be1_ref,
                we2_ref, be2_ref, wx1_ref, bx1_ref, wx2_ref, bx2_ref,
                wh1_ref, bh1_ref, wh2_ref, bh2_ref,
                hout_ref, xout_ref, macc_ref, xacc_ref):
    i = pl.program_id(0)
    j = pl.program_id(1)
    nj = pl.num_programs(1)

    @pl.when(j == 0)
    def _zero():
        macc_ref[...] = jnp.zeros_like(macc_ref)
        xacc_ref[...] = jnp.zeros_like(xacc_ref)

    hi = h_ref[pl.ds(i * bi, bi), :]
    hj = h_ref[pl.ds(j * bj, bj), :]
    xi = x_ref[pl.ds(i * bi, bi), :]
    xj = x_ref[pl.ds(j * bj, bj), :]

    a = jnp.dot(hi, wa_ref[...], preferred_element_type=jnp.float32)
    a = a + be1_ref[...]
    b = jnp.dot(hj, wb_ref[...], preferred_element_type=jnp.float32)

    xi2 = jnp.sum(xi * xi, axis=1, keepdims=True)
    xj2 = jnp.sum(xj * xj, axis=1, keepdims=True)
    cross = jax.lax.dot_general(
        xi, xj, dimension_numbers=(((1,), (1,)), ((), ())),
        preferred_element_type=jnp.float32)
    dist2 = xi2 + xj2.reshape(1, bj) - 2.0 * cross

    t1 = (a[:, None, :] + b[None, :, :]
          + dist2[:, :, None] * wd_ref[...][None, :, :])
    t1 = _silu(t1)

    hdim = t1.shape[-1]
    m = _silu(jnp.dot(t1.reshape(bi * bj, hdim), we2_ref[...],
                      preferred_element_type=jnp.float32) + be2_ref[...])
    wv = _silu(jnp.dot(m, wx1_ref[...],
                       preferred_element_type=jnp.float32) + bx1_ref[...])
    wsc = jnp.dot(wv, wx2_ref[...],
                  preferred_element_type=jnp.float32) + bx2_ref[...]

    gi = i * bi + jax.lax.broadcasted_iota(jnp.int32, (bi, bj), 0)
    gj = j * bj + jax.lax.broadcasted_iota(jnp.int32, (bi, bj), 1)
    mask = (gi != gj).astype(jnp.float32)

    wmat = wsc.reshape(bi, bj) * mask
    m3 = m.reshape(bi, bj, hdim) * mask[:, :, None]
    macc_ref[...] += jnp.sum(m3, axis=1)

    rw = jnp.sum(wmat, axis=1, keepdims=True)
    xacc_ref[...] += rw * xi - jnp.dot(wmat, xj, preferred_element_type=jnp.float32)

    @pl.when(j == nj - 1)
    def _epilogue():
        xout_ref[pl.ds(i * bi, bi), :] = xi + xacc_ref[...] / (n - 1)
        nin = jnp.concatenate([hi, macc_ref[...]], axis=1)
        hh = _silu(jnp.dot(nin, wh1_ref[...],
                           preferred_element_type=jnp.float32) + bh1_ref[...])
        h_new = hi + jnp.dot(hh, wh2_ref[...],
                             preferred_element_type=jnp.float32) + bh2_ref[...]
        hout_ref[pl.ds(i * bi, bi), :] = _silu(h_new)


def _egnn_layer(p, h, x, bi=128, bj=128):
    n, hdim = h.shape
    grid = (n // bi, n // bj)

    we1 = p['We1']
    ops = (
        h, x,
        we1[:hdim], we1[hdim:2 * hdim], we1[2 * hdim:],
        p['be1'].reshape(1, hdim),
        p['We2'], p['be2'].reshape(1, hdim),
        p['Wx1'], p['bx1'].reshape(1, hdim),
        p['Wx2'], p['bx2'].reshape(1, 1),
        p['Wh1'], p['bh1'].reshape(1, hdim),
        p['Wh2'], p['bh2'].reshape(1, hdim),
    )

    def full_spec(arr):
        return pl.BlockSpec(arr.shape, lambda i, j: (0,) * arr.ndim)

    h_new, x_new = pl.pallas_call(
        functools.partial(_layer_body, n, bi, bj),
        grid=grid,
        in_specs=[full_spec(o) for o in ops],
        out_specs=[
            pl.BlockSpec((n, hdim), lambda i, j: (0, 0)),
            pl.BlockSpec((n, 3), lambda i, j: (0, 0)),
        ],
        out_shape=[
            jax.ShapeDtypeStruct((n, hdim), jnp.float32),
            jax.ShapeDtypeStruct((n, 3), jnp.float32),
        ],
        scratch_shapes=[
            pltpu.VMEM((bi, hdim), jnp.float32),
            pltpu.VMEM((bi, 3), jnp.float32),
        ],
        compiler_params=pltpu.CompilerParams(
            dimension_semantics=("arbitrary", "arbitrary")),
        interpret=_INTERPRET,
    )(*ops)
    return h_new, x_new


def _final_body(mu_ref, x_ref, ls_ref, nz_ref, out_ref):
    mu = mu_ref[...] - x_ref[...]
    mu = mu - jnp.mean(mu, axis=0, keepdims=True)
    out_ref[...] = nz_ref[...] * jnp.exp(ls_ref[...]) + mu


def kernel(h, x, params_mu, params_sigma, key):
    hm, xm = h, x
    for p in params_mu:
        hm, xm = _egnn_layer(p, hm, xm)
    hs, xs = h, x
    for p in params_sigma:
        hs, xs = _egnn_layer(p, hs, xs)

    noise = jax.random.normal(jax.random.key(key), x.shape)
    v = pl.pallas_call(
        _final_body,
        out_shape=jax.ShapeDtypeStruct(x.shape, jnp.float32),
        interpret=_INTERPRET,
    )(xm, x, xs, noise)
    return v


# speedup vs baseline: 1.6539x; 1.6539x over previous
import functools

import jax
import jax.numpy as jnp
from jax.experimental import pallas as pl
from jax.experimental.pallas import tpu as pltpu

_INTERPRET = False


def _silu_b(v):
    v = v.astype(jnp.bfloat16)
    return v * (0.5 * jnp.tanh(0.5 * v) + 0.5)


def _silu_f(v):
    return v * (0.5 * jnp.tanh(0.5 * v) + 0.5)


def _pair_layer_body(n, bi, bj, h_ref, xm_ref, xs_ref,
                     wa_ref, wb_ref, wdm_ref, wds_ref, be1_ref,
                     w2_ref, be2_ref, wx1_ref, bx1_ref, wx2_ref, bx2_ref,
                     wh1_ref, bh1_ref, wh2_ref, bh2_ref,
                     hout_ref, xmout_ref, xsout_ref,
                     macc_ref, xam_ref, xas_ref):
    i = pl.program_id(0)
    j = pl.program_id(1)
    nj = pl.num_programs(1)

    @pl.when(j == 0)
    def _zero():
        macc_ref[...] = jnp.zeros_like(macc_ref)
        xam_ref[...] = jnp.zeros_like(xam_ref)
        xas_ref[...] = jnp.zeros_like(xas_ref)

    hi = h_ref[pl.ds(i * bi, bi), :]
    hj = h_ref[pl.ds(j * bj, bj), :]
    xim = xm_ref[pl.ds(i * bi, bi), :]
    xjm = xm_ref[pl.ds(j * bj, bj), :]
    xis = xs_ref[pl.ds(i * bi, bi), :]
    xjs = xs_ref[pl.ds(j * bj, bj), :]

    hib = hi.astype(jnp.bfloat16)
    hjb = hj.astype(jnp.bfloat16)
    a = jnp.dot(hib, wa_ref[...], preferred_element_type=jnp.float32)
    a = a + be1_ref[...]
    b = jnp.dot(hjb, wb_ref[...], preferred_element_type=jnp.float32)

    def dist2(xi, xj):
        xi2 = jnp.sum(xi * xi, axis=1, keepdims=True)
        xj2 = jnp.sum(xj * xj, axis=1, keepdims=True)
        cross = jax.lax.dot_general(
            xi, xj, dimension_numbers=(((1,), (1,)), ((), ())),
            preferred_element_type=jnp.float32)
        return xi2 + xj2.reshape(1, bj) - 2.0 * cross

    d2m = dist2(xim, xjm)
    d2s = dist2(xis, xjs)

    t1 = (a[:, None, :] + b[None, :, :]
          + d2m[:, :, None] * wdm_ref[...][None, :, :]
          + d2s[:, :, None] * wds_ref[...][None, :, :])
    t1b = _silu_b(t1)

    c2 = t1b.shape[-1]
    m_f = jnp.dot(t1b.reshape(bi * bj, c2), w2_ref[...],
                  preferred_element_type=jnp.float32) + be2_ref[...]
    m_b = _silu_b(m_f)
    wv_f = jnp.dot(m_b, wx1_ref[...],
                   preferred_element_type=jnp.float32) + bx1_ref[...]
    wv_b = _silu_b(wv_f)
    wsc = jnp.dot(wv_b, wx2_ref[...],
                  preferred_element_type=jnp.float32) + bx2_ref[...]

    gi = i * bi + jax.lax.broadcasted_iota(jnp.int32, (bi, bj), 0)
    gj = j * bj + jax.lax.broadcasted_iota(jnp.int32, (bi, bj), 1)
    mask = (gi != gj).astype(jnp.float32)

    m3 = m_b.reshape(bi, bj, c2) * mask[:, :, None].astype(jnp.bfloat16)
    macc_ref[...] += jnp.sum(m3, axis=1, dtype=jnp.float32)

    wm = wsc[:, 0:1].reshape(bi, bj) * mask
    ws = wsc[:, 1:2].reshape(bi, bj) * mask
    rwm = jnp.sum(wm, axis=1, keepdims=True)
    rws = jnp.sum(ws, axis=1, keepdims=True)
    xam_ref[...] += rwm * xim - jnp.dot(wm, xjm, preferred_element_type=jnp.float32)
    xas_ref[...] += rws * xis - jnp.dot(ws, xjs, preferred_element_type=jnp.float32)

    @pl.when(j == nj - 1)
    def _epilogue():
        xmout_ref[pl.ds(i * bi, bi), :] = xim + xam_ref[...] / (n - 1)
        xsout_ref[pl.ds(i * bi, bi), :] = xis + xas_ref[...] / (n - 1)
        nin = jnp.concatenate([hi, macc_ref[...]], axis=1)
        hh = _silu_f(jnp.dot(nin, wh1_ref[...],
                             preferred_element_type=jnp.float32) + bh1_ref[...])
        h_new = hi + jnp.dot(hh, wh2_ref[...],
                             preferred_element_type=jnp.float32) + bh2_ref[...]
        hout_ref[pl.ds(i * bi, bi), :] = _silu_f(h_new)


def _block_diag(amu, asig):
    z01 = jnp.zeros((amu.shape[0], asig.shape[1]), jnp.float32)
    z10 = jnp.zeros((asig.shape[0], amu.shape[1]), jnp.float32)
    return jnp.block([[amu, z01], [z10, asig]])


def _pair_layer(pm, ps, hcat, xm, xs, bi=128, bj=128):
    n = hcat.shape[0]
    hd = hcat.shape[1] // 2
    grid = (n // bi, n // bj)
    bf = jnp.bfloat16
    z = jnp.zeros((1, hd), jnp.float32)

    wa = _block_diag(pm['We1'][:hd], ps['We1'][:hd]).astype(bf)
    wb = _block_diag(pm['We1'][hd:2 * hd], ps['We1'][hd:2 * hd]).astype(bf)
    wdm = jnp.concatenate([pm['We1'][2 * hd:], z], axis=1)
    wds = jnp.concatenate([z, ps['We1'][2 * hd:]], axis=1)
    be1 = jnp.concatenate([pm['be1'], ps['be1']]).reshape(1, 2 * hd)
    w2 = _block_diag(pm['We2'], ps['We2']).astype(bf)
    be2 = jnp.concatenate([pm['be2'], ps['be2']]).reshape(1, 2 * hd)
    wx1 = _block_diag(pm['Wx1'], ps['Wx1']).astype(bf)
    bx1 = jnp.concatenate([pm['bx1'], ps['bx1']]).reshape(1, 2 * hd)
    wx2 = _block_diag(pm['Wx2'], ps['Wx2']).astype(bf)
    bx2 = jnp.concatenate([pm['bx2'], ps['bx2']]).reshape(1, 2)
    wh1 = jnp.concatenate([
        _block_diag(pm['Wh1'][:hd], ps['Wh1'][:hd]),
        _block_diag(pm['Wh1'][hd:], ps['Wh1'][hd:]),
    ], axis=0)
    bh1 = jnp.concatenate([pm['bh1'], ps['bh1']]).reshape(1, 2 * hd)
    wh2 = _block_diag(pm['Wh2'], ps['Wh2'])
    bh2 = jnp.concatenate([pm['bh2'], ps['bh2']]).reshape(1, 2 * hd)

    ops = (hcat, xm, xs, wa, wb, wdm, wds, be1, w2, be2,
           wx1, bx1, wx2, bx2, wh1, bh1, wh2, bh2)

    def full_spec(arr):
        return pl.BlockSpec(arr.shape, lambda i, j: (0,) * arr.ndim)

    h_new, xm_new, xs_new = pl.pallas_call(
        functools.partial(_pair_layer_body, n, bi, bj),
        grid=grid,
        in_specs=[full_spec(o) for o in ops],
        out_specs=[
            pl.BlockSpec((n, 2 * hd), lambda i, j: (0, 0)),
            pl.BlockSpec((n, 3), lambda i, j: (0, 0)),
            pl.BlockSpec((n, 3), lambda i, j: (0, 0)),
        ],
        out_shape=[
            jax.ShapeDtypeStruct((n, 2 * hd), jnp.float32),
            jax.ShapeDtypeStruct((n, 3), jnp.float32),
            jax.ShapeDtypeStruct((n, 3), jnp.float32),
        ],
        scratch_shapes=[
            pltpu.VMEM((bi, 2 * hd), jnp.float32),
            pltpu.VMEM((bi, 3), jnp.float32),
            pltpu.VMEM((bi, 3), jnp.float32),
        ],
        compiler_params=pltpu.CompilerParams(
            dimension_semantics=("arbitrary", "arbitrary")),
        interpret=_INTERPRET,
    )(*ops)
    return h_new, xm_new, xs_new


def _final_body(mu_ref, x_ref, ls_ref, nz_ref, out_ref):
    mu = mu_ref[...] - x_ref[...]
    mu = mu - jnp.mean(mu, axis=0, keepdims=True)
    out_ref[...] = nz_ref[...] * jnp.exp(ls_ref[...]) + mu


def kernel(h, x, params_mu, params_sigma, key):
    hcat = jnp.concatenate([h, h], axis=1)
    xm = x
    xs = x
    for pm, ps in zip(params_mu, params_sigma):
        hcat, xm, xs = _pair_layer(pm, ps, hcat, xm, xs)

    noise = jax.random.normal(jax.random.key(key), x.shape)
    v = pl.pallas_call(
        _final_body,
        out_shape=jax.ShapeDtypeStruct(x.shape, jnp.float32),
        interpret=_INTERPRET,
    )(xm, x, xs, noise)
    return v


# speedup vs baseline: 2.2953x; 1.3878x over previous
import functools

import jax
import jax.numpy as jnp
from jax.experimental import pallas as pl
from jax.experimental.pallas import tpu as pltpu

_INTERPRET = False


def _silu_b(v):
    return v * (0.5 * jnp.tanh(0.5 * v) + 0.5)


def _silu_f(v):
    return v * (0.5 * jnp.tanh(0.5 * v) + 0.5)


def _pair_layer_body(n, bi, bj, h_ref, xm_ref, xs_ref,
                     wa_ref, wb_ref, wdm_ref, wds_ref, be1_ref,
                     w2_ref, be2_ref, wx1_ref, bx1_ref, wx2_ref, bx2_ref,
                     wh1_ref, bh1_ref, wh2_ref, bh2_ref,
                     hout_ref, xmout_ref, xsout_ref,
                     macc_ref, xam_ref, xas_ref):
    i = pl.program_id(0)
    j = pl.program_id(1)
    nj = pl.num_programs(1)

    @pl.when(j == 0)
    def _zero():
        macc_ref[...] = jnp.zeros_like(macc_ref)
        xam_ref[...] = jnp.zeros_like(xam_ref)
        xas_ref[...] = jnp.zeros_like(xas_ref)

    hi = h_ref[pl.ds(i * bi, bi), :]
    hj = h_ref[pl.ds(j * bj, bj), :]
    xim = xm_ref[pl.ds(i * bi, bi), :]
    xjm = xm_ref[pl.ds(j * bj, bj), :]
    xis = xs_ref[pl.ds(i * bi, bi), :]
    xjs = xs_ref[pl.ds(j * bj, bj), :]

    bf = jnp.bfloat16
    hib = hi.astype(bf)
    hjb = hj.astype(bf)
    a = jnp.dot(hib, wa_ref[...],
                preferred_element_type=jnp.float32).astype(bf)
    a = a + be1_ref[...]
    b = jnp.dot(hjb, wb_ref[...],
                preferred_element_type=jnp.float32).astype(bf)

    def dist2(xi, xj):
        xi2 = jnp.sum(xi * xi, axis=1, keepdims=True)
        xj2 = jnp.sum(xj * xj, axis=1, keepdims=True)
        cross = jax.lax.dot_general(
            xi, xj, dimension_numbers=(((1,), (1,)), ((), ())),
            preferred_element_type=jnp.float32)
        return (xi2 + xj2.reshape(1, bj) - 2.0 * cross).astype(bf)

    d2m = dist2(xim, xjm)
    d2s = dist2(xis, xjs)

    t1 = (a[:, None, :] + b[None, :, :]
          + d2m[:, :, None] * wdm_ref[...][None, :, :]
          + d2s[:, :, None] * wds_ref[...][None, :, :])
    t1b = _silu_b(t1)

    c2 = t1b.shape[-1]
    m_f = jnp.dot(t1b.reshape(bi * bj, c2), w2_ref[...],
                  preferred_element_type=jnp.float32).astype(bf)
    m_b = _silu_b(m_f + be2_ref[...])
    wv_f = jnp.dot(m_b, wx1_ref[...],
                   preferred_element_type=jnp.float32).astype(bf)
    wv_b = _silu_b(wv_f + bx1_ref[...])
    wsc = jnp.dot(wv_b, wx2_ref[...],
                  preferred_element_type=jnp.float32)

    m3 = m_b.reshape(bi, bj, c2)
    msum = jnp.sum(m3, axis=1, dtype=jnp.float32)

    @pl.when(i != j)
    def _acc_offdiag():
        macc_ref[...] += msum

    @pl.when(i == j)
    def _acc_diag():
        td = _silu_b(a + b)
        md = _silu_b(jnp.dot(td, w2_ref[...],
                             preferred_element_type=jnp.float32).astype(bf)
                     + be2_ref[...])
        macc_ref[...] += msum - md.astype(jnp.float32)

    wm = wsc[:, 0:1].reshape(bi, bj)
    ws = wsc[:, 1:2].reshape(bi, bj)
    rwm = jnp.sum(wm, axis=1, keepdims=True)
    rws = jnp.sum(ws, axis=1, keepdims=True)
    xam_ref[...] += rwm * xim - jnp.dot(wm, xjm, preferred_element_type=jnp.float32)
    xas_ref[...] += rws * xis - jnp.dot(ws, xjs, preferred_element_type=jnp.float32)

    @pl.when(j == nj - 1)
    def _epilogue():
        sxm = jnp.sum(xm_ref[...], axis=0, keepdims=True)
        sxs = jnp.sum(xs_ref[...], axis=0, keepdims=True)
        bm = bx2_ref[0, 0] * (n * xim - sxm)
        bs = bx2_ref[0, 1] * (n * xis - sxs)
        xmout_ref[pl.ds(i * bi, bi), :] = xim + (xam_ref[...] + bm) / (n - 1)
        xsout_ref[pl.ds(i * bi, bi), :] = xis + (xas_ref[...] + bs) / (n - 1)
        nin = jnp.concatenate([hi, macc_ref[...]], axis=1)
        hh = _silu_f(jnp.dot(nin, wh1_ref[...],
                             preferred_element_type=jnp.float32) + bh1_ref[...])
        h_new = hi + jnp.dot(hh, wh2_ref[...],
                             preferred_element_type=jnp.float32) + bh2_ref[...]
        hout_ref[pl.ds(i * bi, bi), :] = _silu_f(h_new)


def _block_diag(amu, asig):
    z01 = jnp.zeros((amu.shape[0], asig.shape[1]), jnp.float32)
    z10 = jnp.zeros((asig.shape[0], amu.shape[1]), jnp.float32)
    return jnp.block([[amu, z01], [z10, asig]])


def _pair_layer(pm, ps, hcat, xm, xs, bi=128, bj=128):
    n = hcat.shape[0]
    hd = hcat.shape[1] // 2
    grid = (n // bi, n // bj)
    bf = jnp.bfloat16
    z = jnp.zeros((1, hd), jnp.float32)

    wa = _block_diag(pm['We1'][:hd], ps['We1'][:hd]).astype(bf)
    wb = _block_diag(pm['We1'][hd:2 * hd], ps['We1'][hd:2 * hd]).astype(bf)
    wdm = jnp.concatenate([pm['We1'][2 * hd:], z], axis=1).astype(bf)
    wds = jnp.concatenate([z, ps['We1'][2 * hd:]], axis=1).astype(bf)
    be1 = jnp.concatenate([pm['be1'], ps['be1']]).reshape(1, 2 * hd).astype(bf)
    w2 = _block_diag(pm['We2'], ps['We2']).astype(bf)
    be2 = jnp.concatenate([pm['be2'], ps['be2']]).reshape(1, 2 * hd).astype(bf)
    wx1 = _block_diag(pm['Wx1'], ps['Wx1']).astype(bf)
    bx1 = jnp.concatenate([pm['bx1'], ps['bx1']]).reshape(1, 2 * hd).astype(bf)
    wx2 = _block_diag(pm['Wx2'], ps['Wx2']).astype(bf)
    bx2 = jnp.concatenate([pm['bx2'], ps['bx2']]).reshape(1, 2)
    wh1 = jnp.concatenate([
        _block_diag(pm['Wh1'][:hd], ps['Wh1'][:hd]),
        _block_diag(pm['Wh1'][hd:], ps['Wh1'][hd:]),
    ], axis=0)
    bh1 = jnp.concatenate([pm['bh1'], ps['bh1']]).reshape(1, 2 * hd)
    wh2 = _block_diag(pm['Wh2'], ps['Wh2'])
    bh2 = jnp.concatenate([pm['bh2'], ps['bh2']]).reshape(1, 2 * hd)

    ops = (hcat, xm, xs, wa, wb, wdm, wds, be1, w2, be2,
           wx1, bx1, wx2, bx2, wh1, bh1, wh2, bh2)

    def full_spec(arr):
        return pl.BlockSpec(arr.shape, lambda i, j: (0,) * arr.ndim)

    h_new, xm_new, xs_new = pl.pallas_call(
        functools.partial(_pair_layer_body, n, bi, bj),
        grid=grid,
        in_specs=[full_spec(o) for o in ops],
        out_specs=[
            pl.BlockSpec((n, 2 * hd), lambda i, j: (0, 0)),
            pl.BlockSpec((n, 3), lambda i, j: (0, 0)),
            pl.BlockSpec((n, 3), lambda i, j: (0, 0)),
        ],
        out_shape=[
            jax.ShapeDtypeStruct((n, 2 * hd), jnp.float32),
            jax.ShapeDtypeStruct((n, 3), jnp.float32),
            jax.ShapeDtypeStruct((n, 3), jnp.float32),
        ],
        scratch_shapes=[
            pltpu.VMEM((bi, 2 * hd), jnp.float32),
            pltpu.VMEM((bi, 3), jnp.float32),
            pltpu.VMEM((bi, 3), jnp.float32),
        ],
        compiler_params=pltpu.CompilerParams(
            dimension_semantics=("arbitrary", "arbitrary")),
        interpret=_INTERPRET,
    )(*ops)
    return h_new, xm_new, xs_new


def _final_body(mu_ref, x_ref, ls_ref, nz_ref, out_ref):
    mu = mu_ref[...] - x_ref[...]
    mu = mu - jnp.mean(mu, axis=0, keepdims=True)
    out_ref[...] = nz_ref[...] * jnp.exp(ls_ref[...]) + mu


def kernel(h, x, params_mu, params_sigma, key):
    hcat = jnp.concatenate([h, h], axis=1)
    xm = x
    xs = x
    for pm, ps in zip(params_mu, params_sigma):
        hcat, xm, xs = _pair_layer(pm, ps, hcat, xm, xs)

    noise = jax.random.normal(jax.random.key(key), x.shape)
    v = pl.pallas_call(
        _final_body,
        out_shape=jax.ShapeDtypeStruct(x.shape, jnp.float32),
        interpret=_INTERPRET,
    )(xm, x, xs, noise)
    return v


# speedup vs baseline: 2.3017x; 1.0028x over previous
import functools

import jax
import jax.numpy as jnp
from jax.experimental import pallas as pl
from jax.experimental.pallas import tpu as pltpu

_INTERPRET = False


def _silu_b(v):
    return v * (0.5 * jnp.tanh(0.5 * v) + 0.5)


def _silu_f(v):
    return v * (0.5 * jnp.tanh(0.5 * v) + 0.5)


def _pair_layer_body(n, bi, bj, h_ref, xm_ref, xs_ref,
                     wa_ref, wb_ref, wdm_ref, wds_ref, be1_ref,
                     w2_ref, be2_ref, wx1_ref, bx1_ref, wx2_ref, bx2_ref,
                     wh1_ref, bh1_ref, wh2_ref, bh2_ref,
                     hout_ref, xmout_ref, xsout_ref,
                     macc_ref, xam_ref, xas_ref):
    i = pl.program_id(0)
    j = pl.program_id(1)
    nj = pl.num_programs(1)

    @pl.when(j == 0)
    def _zero():
        macc_ref[...] = jnp.zeros_like(macc_ref)
        xam_ref[...] = jnp.zeros_like(xam_ref)
        xas_ref[...] = jnp.zeros_like(xas_ref)

    hi = h_ref[pl.ds(i * bi, bi), :]
    hj = h_ref[pl.ds(j * bj, bj), :]
    xim = xm_ref[pl.ds(i * bi, bi), :]
    xjm = xm_ref[pl.ds(j * bj, bj), :]
    xis = xs_ref[pl.ds(i * bi, bi), :]
    xjs = xs_ref[pl.ds(j * bj, bj), :]

    bf = jnp.bfloat16
    hib = hi.astype(bf)
    hjb = hj.astype(bf)
    a = jnp.dot(hib, wa_ref[...],
                preferred_element_type=jnp.float32).astype(bf)
    a = a + be1_ref[...]
    b = jnp.dot(hjb, wb_ref[...],
                preferred_element_type=jnp.float32).astype(bf)

    def dist2(xi, xj):
        xi2 = jnp.sum(xi * xi, axis=1, keepdims=True)
        xj2 = jnp.sum(xj * xj, axis=1, keepdims=True)
        cross = jax.lax.dot_general(
            xi, xj, dimension_numbers=(((1,), (1,)), ((), ())),
            preferred_element_type=jnp.float32)
        return (xi2 + xj2.reshape(1, bj) - 2.0 * cross).astype(bf)

    d2m = dist2(xim, xjm)
    d2s = dist2(xis, xjs)

    t1 = (a[:, None, :] + b[None, :, :]
          + d2m[:, :, None] * wdm_ref[...][None, :, :]
          + d2s[:, :, None] * wds_ref[...][None, :, :])
    t1b = _silu_b(t1)

    c2 = t1b.shape[-1]
    m_f = jnp.dot(t1b.reshape(bi * bj, c2), w2_ref[...],
                  preferred_element_type=jnp.float32).astype(bf)
    m_b = _silu_b(m_f + be2_ref[...])
    wv_f = jnp.dot(m_b, wx1_ref[...],
                   preferred_element_type=jnp.float32).astype(bf)
    wv_b = _silu_b(wv_f + bx1_ref[...])
    wsc = jnp.dot(wv_b, wx2_ref[...],
                  preferred_element_type=jnp.float32)

    m3 = m_b.reshape(bi, bj, c2)
    msum = jnp.sum(m3, axis=1, dtype=jnp.float32)

    @pl.when(i != j)
    def _acc_offdiag():
        macc_ref[...] += msum

    @pl.when(i == j)
    def _acc_diag():
        td = _silu_b(a + b)
        md = _silu_b(jnp.dot(td, w2_ref[...],
                             preferred_element_type=jnp.float32).astype(bf)
                     + be2_ref[...])
        macc_ref[...] += msum - md.astype(jnp.float32)

    wm = wsc[:, 0:1].reshape(bi, bj)
    ws = wsc[:, 1:2].reshape(bi, bj)
    rwm = jnp.sum(wm, axis=1, keepdims=True)
    rws = jnp.sum(ws, axis=1, keepdims=True)
    xam_ref[...] += rwm * xim - jnp.dot(wm, xjm, preferred_element_type=jnp.float32)
    xas_ref[...] += rws * xis - jnp.dot(ws, xjs, preferred_element_type=jnp.float32)

    @pl.when(j == nj - 1)
    def _epilogue():
        sxm = jnp.sum(xm_ref[...], axis=0, keepdims=True)
        sxs = jnp.sum(xs_ref[...], axis=0, keepdims=True)
        bm = bx2_ref[0, 0] * (n * xim - sxm)
        bs = bx2_ref[0, 1] * (n * xis - sxs)
        xmout_ref[...] = xim + (xam_ref[...] + bm) / (n - 1)
        xsout_ref[...] = xis + (xas_ref[...] + bs) / (n - 1)
        nin = jnp.concatenate([hi, macc_ref[...]], axis=1)
        hh = _silu_f(jnp.dot(nin, wh1_ref[...],
                             preferred_element_type=jnp.float32) + bh1_ref[...])
        h_new = hi + jnp.dot(hh, wh2_ref[...],
                             preferred_element_type=jnp.float32) + bh2_ref[...]
        hout_ref[...] = _silu_f(h_new)


def _block_diag(amu, asig):
    z01 = jnp.zeros((amu.shape[0], asig.shape[1]), jnp.float32)
    z10 = jnp.zeros((asig.shape[0], amu.shape[1]), jnp.float32)
    return jnp.block([[amu, z01], [z10, asig]])


def _pair_layer(pm, ps, hcat, xm, xs, bi=128, bj=128):
    n = hcat.shape[0]
    hd = hcat.shape[1] // 2
    grid = (n // bi, n // bj)
    bf = jnp.bfloat16
    z = jnp.zeros((1, hd), jnp.float32)

    wa = _block_diag(pm['We1'][:hd], ps['We1'][:hd]).astype(bf)
    wb = _block_diag(pm['We1'][hd:2 * hd], ps['We1'][hd:2 * hd]).astype(bf)
    wdm = jnp.concatenate([pm['We1'][2 * hd:], z], axis=1).astype(bf)
    wds = jnp.concatenate([z, ps['We1'][2 * hd:]], axis=1).astype(bf)
    be1 = jnp.concatenate([pm['be1'], ps['be1']]).reshape(1, 2 * hd).astype(bf)
    w2 = _block_diag(pm['We2'], ps['We2']).astype(bf)
    be2 = jnp.concatenate([pm['be2'], ps['be2']]).reshape(1, 2 * hd).astype(bf)
    wx1 = _block_diag(pm['Wx1'], ps['Wx1']).astype(bf)
    bx1 = jnp.concatenate([pm['bx1'], ps['bx1']]).reshape(1, 2 * hd).astype(bf)
    wx2 = _block_diag(pm['Wx2'], ps['Wx2']).astype(bf)
    bx2 = jnp.concatenate([pm['bx2'], ps['bx2']]).reshape(1, 2)
    wh1 = jnp.concatenate([
        _block_diag(pm['Wh1'][:hd], ps['Wh1'][:hd]),
        _block_diag(pm['Wh1'][hd:], ps['Wh1'][hd:]),
    ], axis=0)
    bh1 = jnp.concatenate([pm['bh1'], ps['bh1']]).reshape(1, 2 * hd)
    wh2 = _block_diag(pm['Wh2'], ps['Wh2'])
    bh2 = jnp.concatenate([pm['bh2'], ps['bh2']]).reshape(1, 2 * hd)

    ops = (hcat, xm, xs, wa, wb, wdm, wds, be1, w2, be2,
           wx1, bx1, wx2, bx2, wh1, bh1, wh2, bh2)

    def full_spec(arr):
        return pl.BlockSpec(arr.shape, lambda i, j: (0,) * arr.ndim)

    h_new, xm_new, xs_new = pl.pallas_call(
        functools.partial(_pair_layer_body, n, bi, bj),
        grid=grid,
        in_specs=[full_spec(o) for o in ops],
        out_specs=[
            pl.BlockSpec((bi, 2 * hd), lambda i, j: (i, 0)),
            pl.BlockSpec((bi, 3), lambda i, j: (i, 0)),
            pl.BlockSpec((bi, 3), lambda i, j: (i, 0)),
        ],
        out_shape=[
            jax.ShapeDtypeStruct((n, 2 * hd), jnp.float32),
            jax.ShapeDtypeStruct((n, 3), jnp.float32),
            jax.ShapeDtypeStruct((n, 3), jnp.float32),
        ],
        scratch_shapes=[
            pltpu.VMEM((bi, 2 * hd), jnp.float32),
            pltpu.VMEM((bi, 3), jnp.float32),
            pltpu.VMEM((bi, 3), jnp.float32),
        ],
        compiler_params=pltpu.CompilerParams(
            dimension_semantics=("parallel", "arbitrary")),
        interpret=_INTERPRET,
    )(*ops)
    return h_new, xm_new, xs_new


def _final_body(mu_ref, x_ref, ls_ref, nz_ref, out_ref):
    mu = mu_ref[...] - x_ref[...]
    mu = mu - jnp.mean(mu, axis=0, keepdims=True)
    out_ref[...] = nz_ref[...] * jnp.exp(ls_ref[...]) + mu


def kernel(h, x, params_mu, params_sigma, key):
    hcat = jnp.concatenate([h, h], axis=1)
    xm = x
    xs = x
    for pm, ps in zip(params_mu, params_sigma):
        hcat, xm, xs = _pair_layer(pm, ps, hcat, xm, xs)

    noise = jax.random.normal(jax.random.key(key), x.shape)
    v = pl.pallas_call(
        _final_body,
        out_shape=jax.ShapeDtypeStruct(x.shape, jnp.float32),
        interpret=_INTERPRET,
    )(xm, x, xs, noise)
    return v


# speedup vs baseline: 2.4247x; 1.0534x over previous
import functools

import jax
import jax.numpy as jnp
from jax.experimental import pallas as pl
from jax.experimental.pallas import tpu as pltpu

_INTERPRET = False


def _silu(v):
    return v * (0.5 * jnp.tanh(0.5 * v) + 0.5)


def _body(n, bi, bj, depth, *refs):
    bf = jnp.bfloat16
    f32 = jnp.float32
    h_ref, x_ref, nz_ref = refs[0], refs[1], refs[2]
    pnames = ('We1', 'be1', 'We2', 'be2', 'Wx1', 'bx1',
              'Wx2', 'bx2', 'Wh1', 'bh1', 'Wh2', 'bh2')

    def pref(l, mi, name):
        return refs[3 + (l * 2 + mi) * len(pnames) + pnames.index(name)]

    v_ref = refs[3 + depth * 2 * len(pnames)]
    (h_buf, xm_buf, xs_buf, macc, xam, xas,
     wa_s, wb_s, w2_s, wx1_s, wx2_s,
     wdm_s, wds_s, be1_s, be2_s, bx1_s, bx2_s,
     wh1_s, bh1_s, wh2_s, bh2_s) = refs[4 + depth * 2 * len(pnames):]

    l = pl.program_id(0)
    i = pl.program_id(1)
    j = pl.program_id(2)
    nj = pl.num_programs(2)
    hd = h_ref.shape[1]
    c2 = 2 * hd

    @pl.when((l == 0) & (i == 0) & (j == 0))
    def _init():
        h_buf[0, :, 0:hd] = h_ref[...]
        h_buf[0, :, hd:c2] = h_ref[...]
        xm_buf[0] = x_ref[...]
        xs_buf[0] = x_ref[...]
        for ll in range(depth):
            zb = jnp.zeros((hd, hd), bf)
            zf = jnp.zeros((hd, hd), f32)
            zr = jnp.zeros((1, hd), bf)
            for mi in range(2):
                lo, hi_ = mi * hd, (mi + 1) * hd
                oo, oh = (1 - mi) * hd, (2 - mi) * hd
                we1 = pref(ll, mi, 'We1')
                wa_s[ll, lo:hi_, lo:hi_] = we1[0:hd, :].astype(bf)
                wa_s[ll, lo:hi_, oo:oh] = zb
                wb_s[ll, lo:hi_, lo:hi_] = we1[hd:c2, :].astype(bf)
                wb_s[ll, lo:hi_, oo:oh] = zb
                w2_s[ll, lo:hi_, lo:hi_] = pref(ll, mi, 'We2')[...].astype(bf)
                w2_s[ll, lo:hi_, oo:oh] = zb
                wx1_s[ll, lo:hi_, lo:hi_] = pref(ll, mi, 'Wx1')[...].astype(bf)
                wx1_s[ll, lo:hi_, oo:oh] = zb
                wx2_s[ll, lo:hi_, mi:mi + 1] = pref(ll, mi, 'Wx2')[...].astype(bf)
                wx2_s[ll, oo:oh, mi:mi + 1] = jnp.zeros((hd, 1), bf)
                be1_s[ll, :, lo:hi_] = pref(ll, mi, 'be1')[...].astype(bf)
                be2_s[ll, :, lo:hi_] = pref(ll, mi, 'be2')[...].astype(bf)
                bx1_s[ll, :, lo:hi_] = pref(ll, mi, 'bx1')[...].astype(bf)
                bx2_s[ll, :, mi:mi + 1] = pref(ll, mi, 'bx2')[...]
                wh1 = pref(ll, mi, 'Wh1')
                wh1_s[ll, lo:hi_, lo:hi_] = wh1[0:hd, :]
                wh1_s[ll, lo:hi_, oo:oh] = zf
                wh1_s[ll, c2 + lo:c2 + hi_, lo:hi_] = wh1[hd:c2, :]
                wh1_s[ll, c2 + lo:c2 + hi_, oo:oh] = zf
                bh1_s[ll, :, lo:hi_] = pref(ll, mi, 'bh1')[...]
                wh2_s[ll, lo:hi_, lo:hi_] = pref(ll, mi, 'Wh2')[...]
                wh2_s[ll, lo:hi_, oo:oh] = zf
                bh2_s[ll, :, lo:hi_] = pref(ll, mi, 'bh2')[...]
            wdm_s[ll, :, 0:hd] = pref(ll, 0, 'We1')[c2:c2 + 1, :].astype(bf)
            wdm_s[ll, :, hd:c2] = zr
            wds_s[ll, :, 0:hd] = zr
            wds_s[ll, :, hd:c2] = pref(ll, 1, 'We1')[c2:c2 + 1, :].astype(bf)

    @pl.when(j == 0)
    def _zero():
        macc[...] = jnp.zeros_like(macc)
        xam[...] = jnp.zeros_like(xam)
        xas[...] = jnp.zeros_like(xas)

    lr = 1 - l
    hi = h_buf[l, pl.ds(i * bi, bi), :]
    hj = h_buf[l, pl.ds(j * bj, bj), :]
    xim = xm_buf[l, pl.ds(i * bi, bi), :]
    xjm = xm_buf[l, pl.ds(j * bj, bj), :]
    xis = xs_buf[l, pl.ds(i * bi, bi), :]
    xjs = xs_buf[l, pl.ds(j * bj, bj), :]

    a = jnp.dot(hi.astype(bf), wa_s[l],
                preferred_element_type=f32).astype(bf)
    a = a + be1_s[l]
    b = jnp.dot(hj.astype(bf), wb_s[l],
                preferred_element_type=f32).astype(bf)

    def dist2(xi, xj):
        xi2 = jnp.sum(xi * xi, axis=1, keepdims=True)
        xj2 = jnp.sum(xj * xj, axis=1, keepdims=True)
        cross = jax.lax.dot_general(
            xi, xj, dimension_numbers=(((1,), (1,)), ((), ())),
            preferred_element_type=f32)
        return (xi2 + xj2.reshape(1, bj) - 2.0 * cross).astype(bf)

    d2m = dist2(xim, xjm)
    d2s = dist2(xis, xjs)

    t1 = (a[:, None, :] + b[None, :, :]
          + d2m[:, :, None] * wdm_s[l][None, :, :]
          + d2s[:, :, None] * wds_s[l][None, :, :])
    t1b = _silu(t1)

    m_f = jnp.dot(t1b.reshape(bi * bj, c2), w2_s[l],
                  preferred_element_type=f32).astype(bf)
    m_b = _silu(m_f + be2_s[l])
    wv_f = jnp.dot(m_b, wx1_s[l], preferred_element_type=f32).astype(bf)
    wv_b = _silu(wv_f + bx1_s[l])
    wsc = jnp.dot(wv_b, wx2_s[l], preferred_element_type=f32)

    msum = jnp.sum(m_b.reshape(bi, bj, c2), axis=1, dtype=f32)

    @pl.when(i != j)
    def _acc_offdiag():
        macc[...] += msum

    @pl.when(i == j)
    def _acc_diag():
        td = _silu(a + b)
        md = _silu(jnp.dot(td, w2_s[l],
                           preferred_element_type=f32).astype(bf) + be2_s[l])
        macc[...] += msum - md.astype(f32)

    wm = wsc[:, 0:1].reshape(bi, bj)
    ws = wsc[:, 1:2].reshape(bi, bj)
    rwm = jnp.sum(wm, axis=1, keepdims=True)
    rws = jnp.sum(ws, axis=1, keepdims=True)
    xam[...] += rwm * xim - jnp.dot(wm, xjm, preferred_element_type=f32)
    xas[...] += rws * xis - jnp.dot(ws, xjs, preferred_element_type=f32)

    @pl.when(j == nj - 1)
    def _epilogue():
        sxm = jnp.sum(xm_buf[l], axis=0, keepdims=True)
        sxs = jnp.sum(xs_buf[l], axis=0, keepdims=True)
        bx2r = bx2_s[l]
        bm = bx2r[0, 0] * (n * xim - sxm)
        bs = bx2r[0, 1] * (n * xis - sxs)
        xm_buf[lr, pl.ds(i * bi, bi), :] = xim + (xam[...] + bm) / (n - 1)
        xs_buf[lr, pl.ds(i * bi, bi), :] = xis + (xas[...] + bs) / (n - 1)
        nin = jnp.concatenate([hi, macc[...]], axis=1)
        hh = _silu(jnp.dot(nin, wh1_s[l],
                           preferred_element_type=f32) + bh1_s[l])
        h_new = hi + jnp.dot(hh, wh2_s[l],
                             preferred_element_type=f32) + bh2_s[l]
        h_buf[lr, pl.ds(i * bi, bi), :] = _silu(h_new)

    @pl.when((l == depth - 1) & (i == pl.num_programs(1) - 1) & (j == nj - 1))
    def _head():
        mu = xm_buf[lr] - x_ref[...]
        mu = mu - jnp.mean(mu, axis=0, keepdims=True)
        v_ref[...] = nz_ref[...] * jnp.exp(xs_buf[lr]) + mu


def kernel(h, x, params_mu, params_sigma, key):
    n, hd = h.shape
    depth = len(params_mu)
    bi = bj = 128
    bf = jnp.bfloat16
    f32 = jnp.float32
    noise = jax.random.normal(jax.random.key(key), x.shape)

    def shaped(p, name):
        a = p[name]
        return a.reshape(1, -1) if a.ndim == 1 else a

    pnames = ('We1', 'be1', 'We2', 'be2', 'Wx1', 'bx1',
              'Wx2', 'bx2', 'Wh1', 'bh1', 'Wh2', 'bh2')
    ops = [h, x, noise]
    for l in range(depth):
        for p in (params_mu[l], params_sigma[l]):
            ops.extend(shaped(p, name) for name in pnames)

    def full_spec(arr):
        return pl.BlockSpec(arr.shape, lambda l, i, j, nd=arr.ndim: (0,) * nd)

    v = pl.pallas_call(
        functools.partial(_body, n, bi, bj, depth),
        grid=(depth, n // bi, n // bj),
        in_specs=[full_spec(o) for o in ops],
        out_specs=pl.BlockSpec((n, 3), lambda l, i, j: (0, 0)),
        out_shape=jax.ShapeDtypeStruct((n, 3), f32),
        scratch_shapes=[
            pltpu.VMEM((2, n, 2 * hd), f32),
            pltpu.VMEM((2, n, 3), f32),
            pltpu.VMEM((2, n, 3), f32),
            pltpu.VMEM((bi, 2 * hd), f32),
            pltpu.VMEM((bi, 3), f32),
            pltpu.VMEM((bi, 3), f32),
            pltpu.VMEM((depth, 2 * hd, 2 * hd), bf),
            pltpu.VMEM((depth, 2 * hd, 2 * hd), bf),
            pltpu.VMEM((depth, 2 * hd, 2 * hd), bf),
            pltpu.VMEM((depth, 2 * hd, 2 * hd), bf),
            pltpu.VMEM((depth, 2 * hd, 2), bf),
            pltpu.VMEM((depth, 1, 2 * hd), bf),
            pltpu.VMEM((depth, 1, 2 * hd), bf),
            pltpu.VMEM((depth, 1, 2 * hd), bf),
            pltpu.VMEM((depth, 1, 2 * hd), bf),
            pltpu.VMEM((depth, 1, 2 * hd), bf),
            pltpu.VMEM((depth, 1, 2), f32),
            pltpu.VMEM((depth, 4 * hd, 2 * hd), f32),
            pltpu.VMEM((depth, 1, 2 * hd), f32),
            pltpu.VMEM((depth, 2 * hd, 2 * hd), f32),
            pltpu.VMEM((depth, 1, 2 * hd), f32),
        ],
        compiler_params=pltpu.CompilerParams(
            dimension_semantics=("arbitrary", "arbitrary", "arbitrary")),
        interpret=_INTERPRET,
    )(*ops)
    return v
